# Initial kernel scaffold; baseline (speedup 1.0000x reference)
import jax
import jax.numpy as jnp
from jax.experimental import pallas as pl


def kernel(x_nchw, conv1_w, conv1_b, conv2_w, conv2_b, conv3_w, conv3_b, fc1_w, fc1_b, fc2_w, fc2_b):
    raise NotImplementedError("write your pallas kernel here")


# speedup vs baseline: 1.8639x; 1.8639x over previous
import jax
import jax.numpy as jnp
from jax.experimental import pallas as pl
from jax.experimental.pallas import tpu as pltpu

_BB = 64


def _fused_kernel(x_ref, w1t_ref, b1t_ref, w2_ref, b2_ref, w3_ref, b3_ref,
                  wf1_ref, bf1_ref, wf2_ref, bf2_ref, o_ref):
    Bb = x_ref.shape[0]
    x = x_ref[...]

    xcat = jnp.concatenate([x[:, 0:26, :], x[:, 1:27, :], x[:, 2:28, :]],
                           axis=-1)
    y1 = jnp.dot(xcat.reshape(Bb * 26, 84), w1t_ref[...],
                 preferred_element_type=jnp.float32) + b1t_ref[...]
    y1 = jnp.maximum(y1, 0.0)

    y1 = y1.reshape(Bb, 26, 26, 32)
    y1 = y1.reshape(Bb, 13, 2, 26, 32).max(axis=2)
    p1 = y1.reshape(Bb, 13, 13, 2, 32).max(axis=3)

    x2 = jnp.concatenate(
        [p1[:, di:di + 11, dj:dj + 11, :]
         for di in range(3) for dj in range(3)], axis=-1)
    y2 = jnp.dot(x2.reshape(Bb * 121, 288), w2_ref[...],
                 preferred_element_type=jnp.float32) + b2_ref[...]
    y2 = jnp.maximum(y2, 0.0).reshape(Bb, 11, 11, 64)

    y2 = y2[:, 0:10, 0:10, :]
    y2 = y2.reshape(Bb, 5, 2, 10, 64).max(axis=2)
    p2 = y2.reshape(Bb, 5, 5, 2, 64).max(axis=3)

    x3 = jnp.concatenate(
        [p2[:, di:di + 3, dj:dj + 3, :]
         for di in range(3) for dj in range(3)], axis=-1)
    y3 = jnp.dot(x3.reshape(Bb * 9, 576), w3_ref[...],
                 preferred_element_type=jnp.float32) + b3_ref[...]
    y3 = jnp.maximum(y3, 0.0).reshape(Bb, 3, 3, 128)

    f = y3[:, 0:2, 0:2, :].reshape(Bb, 4, 128).max(axis=1)

    h = jnp.maximum(jnp.dot(f, wf1_ref[...],
                            preferred_element_type=jnp.float32)
                    + bf1_ref[...], 0.0)
    logits = (jnp.dot(h, wf2_ref[...], preferred_element_type=jnp.float32)
              + bf2_ref[...])
    m = jnp.max(logits, axis=-1, keepdims=True)
    s = logits - m
    lse = jnp.log(jnp.sum(jnp.exp(s), axis=-1, keepdims=True))
    o_ref[...] = (s - lse).astype(o_ref.dtype)


def kernel(x_nchw, conv1_w, conv1_b, conv2_w, conv2_b, conv3_w, conv3_b,
           fc1_w, fc1_b, fc2_w, fc2_b):
    N, C, H, W = x_nchw.shape
    assert (C, H, W) == (1, 28, 28), (C, H, W)
    Bb = _BB
    n_pad = int(pl.cdiv(N, Bb)) * Bb

    x = x_nchw.reshape(N, 28, 28)
    if n_pad != N:
        x = jnp.pad(x, ((0, n_pad - N), (0, 0), (0, 0)))

    w1r = conv1_w.reshape(3, 3, 32)
    diff = jnp.arange(28)[:, None] - jnp.arange(26)[None, :]
    mask = (diff >= 0) & (diff <= 2)
    idx = jnp.clip(diff, 0, 2)
    w1t = jnp.where(mask[None, :, :, None], w1r[:, idx, :], 0.0)
    w1t = w1t.reshape(84, 26 * 32)
    b1t = jnp.tile(conv1_b.reshape(1, 32), (1, 26))

    w2 = conv2_w.reshape(288, 64)
    b2 = conv2_b.reshape(1, 64)
    w3 = conv3_w.reshape(576, 128)
    b3 = conv3_b.reshape(1, 128)
    wf1 = fc1_w.reshape(128, 9, 512).sum(axis=1)
    bf1 = fc1_b.reshape(1, 512)
    wf2 = fc2_w
    bf2 = fc2_b.reshape(1, 10)

    def const2d(shape):
        return pl.BlockSpec(shape, lambda g: (0, 0))

    out = pl.pallas_call(
        _fused_kernel,
        out_shape=jax.ShapeDtypeStruct((n_pad, 10), jnp.float32),
        grid=(n_pad // Bb,),
        in_specs=[
            pl.BlockSpec((Bb, 28, 28), lambda g: (g, 0, 0)),
            const2d((84, 832)),
            const2d((1, 832)),
            const2d((288, 64)),
            const2d((1, 64)),
            const2d((576, 128)),
            const2d((1, 128)),
            const2d((128, 512)),
            const2d((1, 512)),
            const2d((512, 10)),
            const2d((1, 10)),
        ],
        out_specs=pl.BlockSpec((Bb, 10), lambda g: (g, 0)),
        compiler_params=pltpu.CompilerParams(
            dimension_semantics=("parallel",),
            vmem_limit_bytes=60 * 1024 * 1024,
        ),
    )(x, w1t, b1t, w2, b2, w3, b3, wf1, bf1, wf2, bf2)
    return out[:N]


# speedup vs baseline: 4.3804x; 2.3501x over previous
import jax
import jax.numpy as jnp
from jax.experimental import pallas as pl
from jax.experimental.pallas import tpu as pltpu

_BB = 64


def _fused_kernel(x_ref, w1t_ref, b1t_ref, w2t_ref, b2t_ref, w3t_ref,
                  b3t_ref, wf1_ref, bf1_ref, wf2_ref, bf2_ref, o_ref):
    Bb = x_ref.shape[0]
    x = x_ref[...]

    x1 = jnp.concatenate([x[:, 0:26, :], x[:, 1:27, :], x[:, 2:28, :]],
                         axis=-1)
    y1 = jnp.dot(x1.reshape(Bb * 26, 84), w1t_ref[...],
                 preferred_element_type=jnp.float32) + b1t_ref[...]
    y1 = jnp.maximum(y1, 0.0).reshape(Bb, 26, 832)

    y1 = y1.reshape(Bb, 13, 2, 832).max(axis=2)
    ym = jnp.maximum(y1[..., :800], y1[..., 32:])
    p1 = jnp.concatenate([ym[..., 64 * r:64 * r + 32] for r in range(13)],
                         axis=-1)

    x2 = jnp.concatenate([p1[:, 0:11, :], p1[:, 1:12, :], p1[:, 2:13, :]],
                         axis=-1)
    y2 = jnp.dot(x2.reshape(Bb * 11, 1248), w2t_ref[...],
                 preferred_element_type=jnp.float32) + b2t_ref[...]
    y2 = jnp.maximum(y2, 0.0).reshape(Bb, 11, 704)

    y2 = y2[:, 0:10, :].reshape(Bb, 5, 2, 704).max(axis=2)
    ym2 = jnp.maximum(y2[..., :640], y2[..., 64:])
    p2 = jnp.concatenate([ym2[..., 128 * r:128 * r + 64] for r in range(5)],
                         axis=-1)

    x3 = jnp.concatenate([p2[:, 0:3, :], p2[:, 1:4, :], p2[:, 2:5, :]],
                         axis=-1)
    y3 = jnp.dot(x3.reshape(Bb * 3, 960), w3t_ref[...],
                 preferred_element_type=jnp.float32) + b3t_ref[...]
    y3 = jnp.maximum(y3, 0.0).reshape(Bb, 3, 384)

    yh = jnp.maximum(y3[:, 0, :], y3[:, 1, :])
    f = jnp.maximum(yh[:, :128], yh[:, 128:256])

    h = jnp.maximum(jnp.dot(f, wf1_ref[...],
                            preferred_element_type=jnp.float32)
                    + bf1_ref[...], 0.0)
    logits = (jnp.dot(h, wf2_ref[...], preferred_element_type=jnp.float32)
              + bf2_ref[...])
    m = jnp.max(logits, axis=-1, keepdims=True)
    s = logits - m
    lse = jnp.log(jnp.sum(jnp.exp(s), axis=-1, keepdims=True))
    o_ref[...] = (s - lse).astype(o_ref.dtype)


def _toeplitz_w(w, win, wout, cin, cout):
    diff = jnp.arange(win)[:, None] - jnp.arange(wout)[None, :]
    mask = (diff >= 0) & (diff <= 2)
    idx = jnp.clip(diff, 0, 2)
    wt = w[:, idx]
    wt = jnp.where(mask[None, :, :, None, None], wt, 0.0)
    wt = wt.transpose(0, 1, 3, 2, 4)
    return wt.reshape(3 * win * cin, wout * cout)


def kernel(x_nchw, conv1_w, conv1_b, conv2_w, conv2_b, conv3_w, conv3_b,
           fc1_w, fc1_b, fc2_w, fc2_b):
    N, C, H, W = x_nchw.shape
    assert (C, H, W) == (1, 28, 28), (C, H, W)
    Bb = _BB
    n_pad = int(pl.cdiv(N, Bb)) * Bb

    x = x_nchw.reshape(N, 28, 28)
    if n_pad != N:
        x = jnp.pad(x, ((0, n_pad - N), (0, 0), (0, 0)))

    w1t = _toeplitz_w(conv1_w.reshape(3, 3, 1, 32), 28, 26, 1, 32)
    b1t = jnp.tile(conv1_b.reshape(1, 32), (1, 26))
    w2t = _toeplitz_w(conv2_w, 13, 11, 32, 64)
    b2t = jnp.tile(conv2_b.reshape(1, 64), (1, 11))
    w3t = _toeplitz_w(conv3_w, 5, 3, 64, 128)
    b3t = jnp.tile(conv3_b.reshape(1, 128), (1, 3))
    wf1 = fc1_w.reshape(128, 9, 512).sum(axis=1)
    bf1 = fc1_b.reshape(1, 512)
    wf2 = fc2_w
    bf2 = fc2_b.reshape(1, 10)

    def const2d(shape):
        return pl.BlockSpec(shape, lambda g: (0, 0))

    out = pl.pallas_call(
        _fused_kernel,
        out_shape=jax.ShapeDtypeStruct((n_pad, 10), jnp.float32),
        grid=(n_pad // Bb,),
        in_specs=[
            pl.BlockSpec((Bb, 28, 28), lambda g: (g, 0, 0)),
            const2d((84, 832)),
            const2d((1, 832)),
            const2d((1248, 704)),
            const2d((1, 704)),
            const2d((960, 384)),
            const2d((1, 384)),
            const2d((128, 512)),
            const2d((1, 512)),
            const2d((512, 10)),
            const2d((1, 10)),
        ],
        out_specs=pl.BlockSpec((Bb, 10), lambda g: (g, 0)),
        compiler_params=pltpu.CompilerParams(
            dimension_semantics=("parallel",),
            vmem_limit_bytes=60 * 1024 * 1024,
        ),
    )(x, w1t, b1t, w2t, b2t, w3t, b3t, wf1, bf1, wf2, bf2)
    return out[:N]
